# Initial kernel scaffold; baseline (speedup 1.0000x reference)
import jax
import jax.numpy as jnp
from jax.experimental import pallas as pl


def kernel(inputs):
    raise NotImplementedError("write your pallas kernel here")


# speedup vs baseline: 18.4136x; 18.4136x over previous
import functools

import jax
import jax.numpy as jnp
from jax import lax
from jax.experimental import pallas as pl
from jax.experimental.pallas import tpu as pltpu
from jax.experimental.pallas import tpu_sc as plsc

K = 3
R = 1024
N = 32768
L = 16
CHUNKS = N // L
NC = 2
NS = 16
NW = NC * NS
ROWS_PER_W = R // NW


_GATHER_DNUMS = lax.GatherDimensionNumbers(
    offset_dims=(), collapsed_slice_dims=(0,), start_index_map=(0,))


def _permute(x, pidx):
    return lax.gather(x, pidx[:, None], _GATHER_DNUMS, (1,),
                      mode=lax.GatherScatterMode.PROMISE_IN_BOUNDS)


def _make_kernel():
    mesh = plsc.VectorSubcoreMesh(core_axis_name="c", subcore_axis_name="s")

    @functools.partial(
        pl.kernel,
        mesh=mesh,
        out_type=jax.ShapeDtypeStruct((R, N), jnp.float32),
        scratch_types=[pltpu.VMEM((N,), jnp.float32)],
        compiler_params=pltpu.CompilerParams(needs_layout_passes=False),
    )
    def k(x_hbm, out_hbm, row_v):
        cid = lax.axis_index("c")
        sid = lax.axis_index("s")
        wid = sid * NC + cid
        base = wid * ROWS_PER_W

        neg = jnp.full((L,), -jnp.inf, dtype=jnp.float32)
        zero = jnp.zeros((L,), dtype=jnp.float32)
        one = jnp.ones((L,), dtype=jnp.float32)
        lane = lax.broadcasted_iota(jnp.int32, (L,), 0)
        perms = [jnp.bitwise_xor(lane, d) for d in (1, 2, 4, 8)]

        def allmax(x):
            for p in perms:
                x = jnp.maximum(x, _permute(x, p))
            return x

        def allsum(x):
            for p in perms:
                x = x + _permute(x, p)
            return x

        def do_row(r, carry):
            row = base + r
            pltpu.sync_copy(x_hbm.at[row], row_v)

            def p1(i, ms):
                m1, m2, m3 = ms
                v = row_v[pl.ds(i * L, L)]
                nm1 = jnp.maximum(m1, v)
                l1 = jnp.minimum(m1, v)
                nm2 = jnp.maximum(m2, l1)
                l2 = jnp.minimum(m2, l1)
                nm3 = jnp.maximum(m3, l2)
                return (nm1, nm2, nm3)

            m1, m2, m3 = lax.fori_loop(0, CHUNKS, p1, (neg, neg, neg))

            def cnt3(val):
                c = ((m1 == val).astype(jnp.float32)
                     + (m2 == val).astype(jnp.float32)
                     + (m3 == val).astype(jnp.float32))
                return allsum(c)

            def below3(val):
                w = jnp.maximum(jnp.maximum(
                    jnp.where(m1 < val, m1, neg),
                    jnp.where(m2 < val, m2, neg)),
                    jnp.where(m3 < val, m3, neg))
                return w

            M1 = allmax(m1)
            c1 = cnt3(M1)
            M2 = allmax(below3(M1))
            c2 = cnt3(M2)
            M3 = allmax(below3(M2))
            kth = jnp.where(c1 >= 3.0, M1, jnp.where(c1 + c2 >= 3.0, M2, M3))

            def p2(i, acc):
                v = row_v[pl.ds(i * L, L)]
                kept = jnp.where(v >= kth, v, zero)
                row_v[pl.ds(i * L, L)] = kept
                return acc + kept

            ssum = lax.fori_loop(0, CHUNKS, p2, zero)
            inv = one / allsum(ssum)

            def p3(i, c):
                row_v[pl.ds(i * L, L)] = row_v[pl.ds(i * L, L)] * inv
                return c

            lax.fori_loop(0, CHUNKS, p3, 0)

            pltpu.sync_copy(row_v, out_hbm.at[row])
            return carry

        lax.fori_loop(0, ROWS_PER_W, do_row, 0)

    return k


_sc_kernel = _make_kernel()


def kernel(inputs):
    x = inputs.reshape(R, N)
    out = _sc_kernel(x)
    return out.reshape(inputs.shape)


# speedup vs baseline: 43.1310x; 2.3423x over previous
import functools

import jax
import jax.numpy as jnp
from jax import lax
from jax.experimental import pallas as pl
from jax.experimental.pallas import tpu as pltpu
from jax.experimental.pallas import tpu_sc as plsc

K = 3
R = 1024
N = 32768
L = 16
CHUNKS = N // L
NC = 2
NS = 16
NW = NC * NS
ROWS_PER_W = R // NW
U2 = 4
U3 = 4

_GATHER_DNUMS = lax.GatherDimensionNumbers(
    offset_dims=(), collapsed_slice_dims=(0,), start_index_map=(0,))


def _permute(x, pidx):
    return lax.gather(x, pidx[:, None], _GATHER_DNUMS, (1,),
                      mode=lax.GatherScatterMode.PROMISE_IN_BOUNDS)


def _make_kernel():
    mesh = plsc.VectorSubcoreMesh(core_axis_name="c", subcore_axis_name="s")

    @functools.partial(
        pl.kernel,
        mesh=mesh,
        out_type=jax.ShapeDtypeStruct((R, N), jnp.float32),
        scratch_types=[
            pltpu.VMEM((N,), jnp.float32),
            pltpu.VMEM((N,), jnp.float32),
            pltpu.VMEM((N,), jnp.float32),
            pltpu.SemaphoreType.DMA,
            pltpu.SemaphoreType.DMA,
            pltpu.SemaphoreType.DMA,
        ],
        compiler_params=pltpu.CompilerParams(needs_layout_passes=False),
    )
    def k(x_hbm, out_hbm, in_a, in_b, stage, sem_a, sem_b, sem_o):
        cid = lax.axis_index("c")
        sid = lax.axis_index("s")
        wid = sid * NC + cid
        base = wid * ROWS_PER_W

        neg = jnp.full((L,), -jnp.inf, dtype=jnp.float32)
        zero = jnp.zeros((L,), dtype=jnp.float32)
        one = jnp.ones((L,), dtype=jnp.float32)
        lane = lax.broadcasted_iota(jnp.int32, (L,), 0)
        perms = [jnp.bitwise_xor(lane, d) for d in (1, 2, 4, 8)]

        def allmax(x):
            for p in perms:
                x = jnp.maximum(x, _permute(x, p))
            return x

        def allsum(x):
            for p in perms:
                x = x + _permute(x, p)
            return x

        def compute_kth(buf):
            def p1(i, ms):
                m1, m2, m3 = ms
                v = buf[pl.ds(i * L, L)]
                nm1 = jnp.maximum(m1, v)
                l1 = jnp.minimum(m1, v)
                nm2 = jnp.maximum(m2, l1)
                l2 = jnp.minimum(m2, l1)
                nm3 = jnp.maximum(m3, l2)
                return (nm1, nm2, nm3)

            m1, m2, m3 = lax.fori_loop(0, CHUNKS, p1, (neg, neg, neg))

            def cnt3(val):
                c = ((m1 == val).astype(jnp.float32)
                     + (m2 == val).astype(jnp.float32)
                     + (m3 == val).astype(jnp.float32))
                return allsum(c)

            def below3(val):
                return jnp.maximum(jnp.maximum(
                    jnp.where(m1 < val, m1, neg),
                    jnp.where(m2 < val, m2, neg)),
                    jnp.where(m3 < val, m3, neg))

            M1 = allmax(m1)
            c1 = cnt3(M1)
            M2 = allmax(below3(M1))
            c2 = cnt3(M2)
            M3 = allmax(below3(M2))
            return jnp.where(c1 >= 3.0, M1, jnp.where(c1 + c2 >= 3.0, M2, M3))

        def mask_sum(buf, kth):
            def p2(i, accs):
                accs = list(accs)
                for u in range(U2):
                    j = i * U2 + u
                    v = buf[pl.ds(j * L, L)]
                    kept = jnp.where(v >= kth, v, zero)
                    stage[pl.ds(j * L, L)] = kept
                    accs[u] = accs[u] + kept
                return tuple(accs)

            accs = lax.fori_loop(0, CHUNKS // U2, p2, (zero,) * U2)
            ssum = (accs[0] + accs[1]) + (accs[2] + accs[3])
            return one / allsum(ssum)

        def scale(inv):
            def p3(i, c):
                for u in range(U3):
                    j = i * U3 + u
                    stage[pl.ds(j * L, L)] = stage[pl.ds(j * L, L)] * inv
                return c

            lax.fori_loop(0, CHUNKS // U3, p3, 0)

        def do_row(buf, sem, row, first_out, nxt_row, do_prefetch):
            kth = compute_kth(buf)
            @pl.when(jnp.logical_not(first_out))
            def _():
                pltpu.make_async_copy(stage, out_hbm.at[0], sem_o).wait()
            inv = mask_sum(buf, kth)
            scale(inv)
            pltpu.async_copy(stage, out_hbm.at[row], sem_o)
            @pl.when(do_prefetch)
            def _():
                pltpu.async_copy(x_hbm.at[nxt_row], buf, sem)

        pltpu.async_copy(x_hbm.at[base], in_a, sem_a)
        pltpu.async_copy(x_hbm.at[base + 1], in_b, sem_b)

        def pair(j, carry):
            r0 = base + 2 * j
            r1 = r0 + 1
            pltpu.make_async_copy(x_hbm.at[0], in_a, sem_a).wait()
            do_row(in_a, sem_a, r0, j == 0, r0 + 2,
                   j < (ROWS_PER_W // 2 - 1))
            pltpu.make_async_copy(x_hbm.at[0], in_b, sem_b).wait()
            do_row(in_b, sem_b, r1, False, r1 + 2,
                   j < (ROWS_PER_W // 2 - 1))
            return carry

        lax.fori_loop(0, ROWS_PER_W // 2, pair, 0)
        pltpu.make_async_copy(stage, out_hbm.at[0], sem_o).wait()

    return k


_sc_kernel = _make_kernel()


def kernel(inputs):
    x = inputs.reshape(R, N)
    out = _sc_kernel(x)
    return out.reshape(inputs.shape)


# speedup vs baseline: 88.0480x; 2.0414x over previous
import functools

import jax
import jax.numpy as jnp
from jax import lax
from jax.experimental import pallas as pl
from jax.experimental.pallas import tpu as pltpu
from jax.experimental.pallas import tpu_sc as plsc

K = 3
R = 1024
N = 32768
L = 16
NSEG = 16
SEGN = N // NSEG
SEGC = SEGN // L
NC = 2
NS = 16
NW = NC * NS
ROWS_PER_W = R // NW

_GATHER_DNUMS = lax.GatherDimensionNumbers(
    offset_dims=(), collapsed_slice_dims=(0,), start_index_map=(0,))


def _permute(x, pidx):
    return lax.gather(x, pidx[:, None], _GATHER_DNUMS, (1,),
                      mode=lax.GatherScatterMode.PROMISE_IN_BOUNDS)


def _make_kernel():
    mesh = plsc.VectorSubcoreMesh(core_axis_name="c", subcore_axis_name="s")

    @functools.partial(
        pl.kernel,
        mesh=mesh,
        out_type=jax.ShapeDtypeStruct((R, N), jnp.float32),
        scratch_types=[
            pltpu.VMEM((N,), jnp.float32),
            pltpu.VMEM((N,), jnp.float32),
            pltpu.VMEM((N,), jnp.float32),
            pltpu.SemaphoreType.DMA,
            pltpu.SemaphoreType.DMA,
            pltpu.SemaphoreType.DMA,
        ],
        compiler_params=pltpu.CompilerParams(needs_layout_passes=False),
    )
    def k(x_hbm, out_hbm, in_a, in_b, stage, sem_a, sem_b, sem_o):
        cid = lax.axis_index("c")
        sid = lax.axis_index("s")
        wid = sid * NC + cid
        base = wid * ROWS_PER_W

        neg = jnp.full((L,), -jnp.inf, dtype=jnp.float32)
        zero = jnp.zeros((L,), dtype=jnp.float32)
        one = jnp.ones((L,), dtype=jnp.float32)
        lane = lax.broadcasted_iota(jnp.int32, (L,), 0)
        perms = [jnp.bitwise_xor(lane, d) for d in (1, 2, 4, 8)]

        def allmax(x):
            for p in perms:
                x = jnp.maximum(x, _permute(x, p))
            return x

        def allsum(x):
            for p in perms:
                x = x + _permute(x, p)
            return x

        def lane_scalar(vec, t):
            return _permute(vec, jnp.full((L,), t, dtype=jnp.int32))[0]

        def seg_maxes(buf):
            def one_seg(s, segvec):
                sbase = s * SEGN

                def body(i, accs):
                    a0, a1, a2, a3 = accs
                    j = sbase + i * 4 * L
                    v0 = buf[pl.ds(j, L)]
                    v1 = buf[pl.ds(j + L, L)]
                    v2 = buf[pl.ds(j + 2 * L, L)]
                    v3 = buf[pl.ds(j + 3 * L, L)]
                    return (jnp.maximum(a0, v0), jnp.maximum(a1, v1),
                            jnp.maximum(a2, v2), jnp.maximum(a3, v3))

                a0, a1, a2, a3 = lax.fori_loop(0, SEGC // 4, body, (neg,) * 4)
                sm = allmax(jnp.maximum(jnp.maximum(a0, a1),
                                        jnp.maximum(a2, a3)))
                return jnp.where(lane == s, sm, segvec)

            return lax.fori_loop(0, NSEG, one_seg, neg)

        def topk_threshold(buf, ids):
            def scan_seg(ms, sbase):
                def body(i, ms2):
                    m1, m2, m3 = ms2
                    v = buf[pl.ds(sbase + i * L, L)]
                    nm1 = jnp.maximum(m1, v)
                    l1 = jnp.minimum(m1, v)
                    nm2 = jnp.maximum(m2, l1)
                    l2 = jnp.minimum(m2, l1)
                    nm3 = jnp.maximum(m3, l2)
                    return (nm1, nm2, nm3)

                return lax.fori_loop(0, SEGC, body, ms)

            ms = (neg, neg, neg)
            for t in range(K):
                ms = scan_seg(ms, lane_scalar(ids, t) * SEGN)
            m1, m2, m3 = ms

            def cnt3(val):
                c = ((m1 == val).astype(jnp.float32)
                     + (m2 == val).astype(jnp.float32)
                     + (m3 == val).astype(jnp.float32))
                return allsum(c)

            def below3(val):
                return jnp.maximum(jnp.maximum(
                    jnp.where(m1 < val, m1, neg),
                    jnp.where(m2 < val, m2, neg)),
                    jnp.where(m3 < val, m3, neg))

            M1 = allmax(m1)
            c1 = cnt3(M1)
            M2 = allmax(below3(M1))
            c2 = cnt3(M2)
            M3 = allmax(below3(M2))
            return jnp.where(c1 >= 3.0, M1, jnp.where(c1 + c2 >= 3.0, M2, M3))

        def do_row(buf, sem, row, first_out, nxt_row, do_prefetch,
                   prev_ids, np_prev):
            segvec = seg_maxes(buf)
            _, ids = plsc.sort_key_val(segvec, lane, descending=True)
            kth = topk_threshold(buf, ids)

            nseg = allsum((segvec >= kth).astype(jnp.float32))[0]
            nseg = nseg.astype(jnp.int32)

            def seg_sum(t, acc):
                sbase = lane_scalar(ids, t) * SEGN

                def body(i, accs):
                    a0, a1 = accs
                    j = sbase + i * 2 * L
                    v0 = buf[pl.ds(j, L)]
                    v1 = buf[pl.ds(j + L, L)]
                    a0 = a0 + jnp.where(v0 >= kth, v0, zero)
                    a1 = a1 + jnp.where(v1 >= kth, v1, zero)
                    return (a0, a1)

                s0, s1 = lax.fori_loop(0, SEGC // 2, body, (acc, zero))
                return s0 + s1

            ssum = lax.fori_loop(0, nseg, seg_sum, zero)
            inv = one / allsum(ssum)

            @pl.when(jnp.logical_not(first_out))
            def _():
                pltpu.make_async_copy(stage, out_hbm.at[0], sem_o).wait()

            def seg_zero(t, c):
                sbase = lane_scalar(prev_ids, t) * SEGN

                def body(i, c2):
                    j = sbase + i * 4 * L
                    stage[pl.ds(j, L)] = zero
                    stage[pl.ds(j + L, L)] = zero
                    stage[pl.ds(j + 2 * L, L)] = zero
                    stage[pl.ds(j + 3 * L, L)] = zero
                    return c2

                return lax.fori_loop(0, SEGC // 4, body, c)

            lax.fori_loop(0, np_prev, seg_zero, 0)

            def seg_write(t, c):
                sbase = lane_scalar(ids, t) * SEGN

                def body(i, c2):
                    j = sbase + i * 2 * L
                    v0 = buf[pl.ds(j, L)]
                    v1 = buf[pl.ds(j + L, L)]
                    stage[pl.ds(j, L)] = jnp.where(v0 >= kth, v0, zero) * inv
                    stage[pl.ds(j + L, L)] = jnp.where(v1 >= kth, v1, zero) * inv
                    return c2

                return lax.fori_loop(0, SEGC // 2, body, c)

            lax.fori_loop(0, nseg, seg_write, 0)

            pltpu.async_copy(stage, out_hbm.at[row], sem_o)

            @pl.when(do_prefetch)
            def _():
                pltpu.async_copy(x_hbm.at[nxt_row], buf, sem)

            return ids, nseg

        def zinit(i, c):
            j = i * 4 * L
            stage[pl.ds(j, L)] = zero
            stage[pl.ds(j + L, L)] = zero
            stage[pl.ds(j + 2 * L, L)] = zero
            stage[pl.ds(j + 3 * L, L)] = zero
            return c

        lax.fori_loop(0, N // (4 * L), zinit, 0)

        pltpu.async_copy(x_hbm.at[base], in_a, sem_a)
        pltpu.async_copy(x_hbm.at[base + 1], in_b, sem_b)

        def pair(j, carry):
            prev_ids, np_prev = carry
            r0 = base + 2 * j
            r1 = r0 + 1
            pltpu.make_async_copy(x_hbm.at[0], in_a, sem_a).wait()
            ids0, np0 = do_row(in_a, sem_a, r0, j == 0, r0 + 2,
                               j < (ROWS_PER_W // 2 - 1), prev_ids, np_prev)
            pltpu.make_async_copy(x_hbm.at[0], in_b, sem_b).wait()
            ids1, np1 = do_row(in_b, sem_b, r1, False, r1 + 2,
                               j < (ROWS_PER_W // 2 - 1), ids0, np0)
            return (ids1, np1)

        lax.fori_loop(0, ROWS_PER_W // 2, pair, (lane, jnp.int32(0)))
        pltpu.make_async_copy(stage, out_hbm.at[0], sem_o).wait()

    return k


_sc_kernel = _make_kernel()


def kernel(inputs):
    x = inputs.reshape(R, N)
    out = _sc_kernel(x)
    return out.reshape(inputs.shape)


# speedup vs baseline: 123.5207x; 1.4029x over previous
import functools

import jax
import jax.numpy as jnp
from jax import lax
from jax.experimental import pallas as pl
from jax.experimental.pallas import tpu as pltpu
from jax.experimental.pallas import tpu_sc as plsc

K = 3
R = 1024
N = 32768
L = 16
NSEG = 16
SEGN = N // NSEG
SEGC = SEGN // L
NC = 2
NS = 16
NW = NC * NS
ROWS_PER_W = R // NW

_GATHER_DNUMS = lax.GatherDimensionNumbers(
    offset_dims=(), collapsed_slice_dims=(0,), start_index_map=(0,))


def _permute(x, pidx):
    return lax.gather(x, pidx[:, None], _GATHER_DNUMS, (1,),
                      mode=lax.GatherScatterMode.PROMISE_IN_BOUNDS)


def _make_kernel():
    mesh = plsc.VectorSubcoreMesh(core_axis_name="c", subcore_axis_name="s")

    @functools.partial(
        pl.kernel,
        mesh=mesh,
        out_type=jax.ShapeDtypeStruct((R, N), jnp.float32),
        scratch_types=[
            pltpu.VMEM((N,), jnp.float32),
            pltpu.VMEM((N,), jnp.float32),
            pltpu.VMEM((N,), jnp.float32),
            pltpu.VMEM((SEGN,), jnp.float32),
            pltpu.SemaphoreType.DMA,
            pltpu.SemaphoreType.DMA,
            pltpu.SemaphoreType.DMA,
        ],
        compiler_params=pltpu.CompilerParams(needs_layout_passes=False),
    )
    def k(x_hbm, out_hbm, in_a, in_b, stage, zbuf, sem_a, sem_b, sem_o):
        cid = lax.axis_index("c")
        sid = lax.axis_index("s")
        wid = sid * NC + cid
        base = wid * ROWS_PER_W

        neg = jnp.full((L,), -jnp.inf, dtype=jnp.float32)
        zero = jnp.zeros((L,), dtype=jnp.float32)
        one = jnp.ones((L,), dtype=jnp.float32)
        lane = lax.broadcasted_iota(jnp.int32, (L,), 0)
        perms = [jnp.bitwise_xor(lane, d) for d in (1, 2, 4, 8)]

        def allmax(x):
            for p in perms:
                x = jnp.maximum(x, _permute(x, p))
            return x

        def allsum(x):
            for p in perms:
                x = x + _permute(x, p)
            return x

        def lane_scalar(vec, t):
            return _permute(vec, jnp.full((L,), t, dtype=jnp.int32))[0]

        def seg_maxes(buf):
            def one_seg(s, segvec):
                sbase = s * SEGN

                def body(i, accs):
                    j = sbase + i * 8 * L
                    return tuple(
                        jnp.maximum(accs[u], buf[pl.ds(j + u * L, L)])
                        for u in range(8))

                a = lax.fori_loop(0, SEGC // 8, body, (neg,) * 8)
                m01 = jnp.maximum(jnp.maximum(a[0], a[1]),
                                  jnp.maximum(a[2], a[3]))
                m23 = jnp.maximum(jnp.maximum(a[4], a[5]),
                                  jnp.maximum(a[6], a[7]))
                sm = allmax(jnp.maximum(m01, m23))
                return jnp.where(lane == s, sm, segvec)

            return lax.fori_loop(0, NSEG, one_seg, neg)

        def insert3(ms, v):
            m1, m2, m3 = ms
            nm1 = jnp.maximum(m1, v)
            l1 = jnp.minimum(m1, v)
            nm2 = jnp.maximum(m2, l1)
            l2 = jnp.minimum(m2, l1)
            nm3 = jnp.maximum(m3, l2)
            return (nm1, nm2, nm3)

        def topk_threshold(buf, ids):
            def scan_seg(ms6, sbase):
                def body(i, ms):
                    ma, mb = ms
                    j = sbase + i * 2 * L
                    va = buf[pl.ds(j, L)]
                    vb = buf[pl.ds(j + L, L)]
                    return (insert3(ma, va), insert3(mb, vb))

                return lax.fori_loop(0, SEGC // 2, body, ms6)

            ms6 = ((neg, neg, neg), (neg, neg, neg))
            for t in range(K):
                ms6 = scan_seg(ms6, lane_scalar(ids, t) * SEGN)
            ma, mb = ms6
            for v in mb:
                ma = insert3(ma, v)
            m1, m2, m3 = ma

            def cnt3(val):
                c = ((m1 == val).astype(jnp.float32)
                     + (m2 == val).astype(jnp.float32)
                     + (m3 == val).astype(jnp.float32))
                return allsum(c)

            def below3(val):
                return jnp.maximum(jnp.maximum(
                    jnp.where(m1 < val, m1, neg),
                    jnp.where(m2 < val, m2, neg)),
                    jnp.where(m3 < val, m3, neg))

            M1 = allmax(m1)
            c1 = cnt3(M1)
            M2 = allmax(below3(M1))
            c2 = cnt3(M2)
            M3 = allmax(below3(M2))
            return jnp.where(c1 >= 3.0, M1, jnp.where(c1 + c2 >= 3.0, M2, M3))

        def do_row(buf, sem, row, first_out, nxt_row, do_prefetch):
            segvec = seg_maxes(buf)
            _, ids = plsc.sort_key_val(segvec, lane, descending=True)
            kth = topk_threshold(buf, ids)

            flags = (segvec >= kth).astype(jnp.float32)
            nseg = allsum(flags)[0].astype(jnp.int32)

            def seg_sum(t, acc):
                sbase = lane_scalar(ids, t) * SEGN

                def body(i, accs):
                    a0, a1, a2, a3 = accs
                    j = sbase + i * 4 * L
                    v0 = buf[pl.ds(j, L)]
                    v1 = buf[pl.ds(j + L, L)]
                    v2 = buf[pl.ds(j + 2 * L, L)]
                    v3 = buf[pl.ds(j + 3 * L, L)]
                    return (a0 + jnp.where(v0 >= kth, v0, zero),
                            a1 + jnp.where(v1 >= kth, v1, zero),
                            a2 + jnp.where(v2 >= kth, v2, zero),
                            a3 + jnp.where(v3 >= kth, v3, zero))

                s0, s1, s2, s3 = lax.fori_loop(0, SEGC // 4, body,
                                               (acc, zero, zero, zero))
                return (s0 + s1) + (s2 + s3)

            ssum = lax.fori_loop(0, nseg, seg_sum, zero)
            inv = one / allsum(ssum)

            @pl.when(jnp.logical_not(first_out))
            def _():
                for _s in range(NSEG):
                    pltpu.make_async_copy(
                        zbuf, out_hbm.at[0, pl.ds(0, SEGN)], sem_o).wait()

            def seg_write(t, c):
                sbase = lane_scalar(ids, t) * SEGN

                def body(i, c2):
                    j = sbase + i * 4 * L
                    for u in range(4):
                        v = buf[pl.ds(j + u * L, L)]
                        stage[pl.ds(j + u * L, L)] = (
                            jnp.where(v >= kth, v, zero) * inv)
                    return c2

                return lax.fori_loop(0, SEGC // 4, body, c)

            lax.fori_loop(0, nseg, seg_write, 0)

            kth0 = kth[0]
            for s in range(NSEG):
                smax = lane_scalar(segvec, s)
                sl = pl.ds(s * SEGN, SEGN)

                @pl.when(smax >= kth0)
                def _(sl=sl):
                    pltpu.async_copy(stage.at[sl], out_hbm.at[row, sl], sem_o)

                @pl.when(smax < kth0)
                def _(sl=sl):
                    pltpu.async_copy(zbuf, out_hbm.at[row, sl], sem_o)

            @pl.when(do_prefetch)
            def _():
                pltpu.async_copy(x_hbm.at[nxt_row], buf, sem)

        def zinit(i, c):
            j = i * 4 * L
            for u in range(4):
                zbuf[pl.ds(j + u * L, L)] = zero
            return c

        lax.fori_loop(0, SEGC // 4, zinit, 0)

        pltpu.async_copy(x_hbm.at[base], in_a, sem_a)
        pltpu.async_copy(x_hbm.at[base + 1], in_b, sem_b)

        def pair(j, carry):
            r0 = base + 2 * j
            r1 = r0 + 1
            pltpu.make_async_copy(x_hbm.at[0], in_a, sem_a).wait()
            do_row(in_a, sem_a, r0, j == 0, r0 + 2,
                   j < (ROWS_PER_W // 2 - 1))
            pltpu.make_async_copy(x_hbm.at[0], in_b, sem_b).wait()
            do_row(in_b, sem_b, r1, False, r1 + 2,
                   j < (ROWS_PER_W // 2 - 1))
            return carry

        lax.fori_loop(0, ROWS_PER_W // 2, pair, 0)
        for _s in range(NSEG):
            pltpu.make_async_copy(
                zbuf, out_hbm.at[0, pl.ds(0, SEGN)], sem_o).wait()

    return k


_sc_kernel = _make_kernel()


def kernel(inputs):
    x = inputs.reshape(R, N)
    out = _sc_kernel(x)
    return out.reshape(inputs.shape)
